# Initial kernel scaffold; baseline (speedup 1.0000x reference)
import jax
import jax.numpy as jnp
from jax.experimental import pallas as pl


def kernel(x, src_trg):
    raise NotImplementedError("write your pallas kernel here")


# speedup vs baseline: 2.0020x; 2.0020x over previous
import functools

import jax
import jax.numpy as jnp
from jax import lax
from jax.experimental import pallas as pl
from jax.experimental.pallas import tpu as pltpu
from jax.experimental.pallas import tpu_sc as plsc

N = 10000
M = 160000
D = 256
K = 8
DD = D // K

NW = 32
EPT = M // NW
E = 40
NCH = EPT // E
G = 8


def _norm_body(x_ref, o_ref):
    xb = x_ref[...]
    sq = xb * xb
    row = lax.broadcasted_iota(jnp.int32, (D, K), 0)
    col = lax.broadcasted_iota(jnp.int32, (D, K), 1)
    seg = (row // DD == col).astype(jnp.float32)
    s = lax.dot_general(sq, seg, (((1,), (0,)), ((), ())),
                        preferred_element_type=jnp.float32)
    inv = 1.0 / jnp.maximum(jnp.sqrt(s), 1e-12)
    invf = lax.dot_general(inv, seg.T, (((1,), (0,)), ((), ())),
                           preferred_element_type=jnp.float32)
    o_ref[...] = xb * invf


def _normalize(x):
    B = 1000
    return pl.pallas_call(
        _norm_body,
        grid=(N // B,),
        in_specs=[pl.BlockSpec((B, D), lambda i: (i, 0))],
        out_specs=pl.BlockSpec((B, D), lambda i: (i, 0)),
        out_shape=jax.ShapeDtypeStruct((N, D), jnp.float32),
    )(x)


def _route_body(xn_hbm, src_hbm, trg_hbm, out_hbm,
                src_v, trg_v, zs, zt, po, sem_s, sem_t):
    wid = lax.axis_index("s") * 2 + lax.axis_index("c")
    pltpu.sync_copy(src_hbm.at[wid], src_v)
    pltpu.sync_copy(trg_hbm.at[wid], trg_v)
    lane = lax.iota(jnp.int32, 16)
    low = lane < G

    def chunk_body(c, _):
        cs = pltpu.async_copy(xn_hbm.at[src_v.at[pl.ds(c * E, E)]], zs, sem_s)
        ct = pltpu.async_copy(xn_hbm.at[trg_v.at[pl.ds(c * E, E)]], zt, sem_t)
        cs.wait()
        ct.wait()

        def group_body(g, _):
            eb = g * G
            pk = [jnp.zeros((16,), jnp.float32) for _ in range(K)]
            for el in range(G):
                e = eb + el
                for kk in range(K):
                    a0 = zs[e, pl.ds(kk * DD, 16)]
                    b0 = zt[e, pl.ds(kk * DD, 16)]
                    a1 = zs[e, pl.ds(kk * DD + 16, 16)]
                    b1 = zt[e, pl.ds(kk * DD + 16, 16)]
                    dot = plsc.cumsum(a0 * b0 + a1 * b1)[15]
                    pk[kk] = jnp.where(lane == el, dot, pk[kk])
            mx = pk[0]
            for kk in range(1, K):
                mx = jnp.maximum(mx, pk[kk])
            ex = [jnp.exp(p - mx) for p in pk]
            tot = ex[0]
            for kk in range(1, K):
                tot = tot + ex[kk]
            inv = 1.0 / tot
            flat = (lane + (c * E + eb)) * K
            for kk in range(K):
                plsc.store_scatter(po, [flat + kk], ex[kk] * inv, mask=low)
            return ()

        lax.fori_loop(0, E // G, group_body, ())
        return ()

    lax.fori_loop(0, NCH, chunk_body, ())
    pltpu.sync_copy(po, out_hbm.at[pl.ds(wid * EPT * K, EPT * K)])


_route = pl.kernel(
    _route_body,
    out_type=jax.ShapeDtypeStruct((M * K,), jnp.float32),
    mesh=plsc.VectorSubcoreMesh(core_axis_name="c", subcore_axis_name="s"),
    compiler_params=pltpu.CompilerParams(needs_layout_passes=False,
                                         use_tc_tiling_on_sc=False),
    scratch_types=[
        pltpu.VMEM((EPT,), jnp.int32),
        pltpu.VMEM((EPT,), jnp.int32),
        pltpu.VMEM((E, D), jnp.float32),
        pltpu.VMEM((E, D), jnp.float32),
        pltpu.VMEM((EPT * K,), jnp.float32),
        pltpu.SemaphoreType.DMA,
        pltpu.SemaphoreType.DMA,
    ],
)


def kernel(x, src_trg):
    xn = _normalize(x)
    src = src_trg[0].reshape(NW, EPT)
    trg = src_trg[1].reshape(NW, EPT)
    return _route(xn, src, trg).reshape(M, K)
